# baseline (speedup 1.0000x reference)
import functools

import jax
import jax.numpy as jnp
from jax import lax
from jax.experimental import pallas as pl
from jax.experimental.pallas import tpu as pltpu
from jax.experimental.pallas import tpu_sc as plsc

_B = 1024
_D = 32
_HID = 128
_TILE_V = 2048


def _sc_gather4(item_id, user_segment, region, device_type,
                E_item, E_seg, E_region, E_device):
  info = plsc.get_sparse_core_info()
  nc, ns = info.num_cores, info.num_subcores
  nw = nc * ns
  bpw = _B // nw

  mesh = plsc.VectorSubcoreMesh(core_axis_name="c", subcore_axis_name="s")
  out_t = [jax.ShapeDtypeStruct((_B, _D), jnp.float32) for _ in range(4)]

  @functools.partial(
      pl.kernel,
      out_type=out_t,
      mesh=mesh,
      compiler_params=pltpu.CompilerParams(use_tc_tiling_on_sc=False),
      scratch_types=[
          pltpu.VMEM((4, bpw), jnp.int32),
          pltpu.VMEM((4, bpw, _D), jnp.float32),
          pltpu.SemaphoreType.DMA,
          pltpu.SemaphoreType.DMA,
          pltpu.SemaphoreType.DMA,
          pltpu.SemaphoreType.DMA,
      ],
  )
  def k(idx0, idx1, idx2, idx3, t0, t1, t2, t3,
        o0, o1, o2, o3, idx_v, rows_v, s0, s1, s2, s3):
    wid = lax.axis_index("s") * nc + lax.axis_index("c")
    base = wid * bpw
    idxs = (idx0, idx1, idx2, idx3)
    tabs = (t0, t1, t2, t3)
    outs = (o0, o1, o2, o3)
    sems = (s0, s1, s2, s3)
    for v in range(4):
      pltpu.sync_copy(idxs[v].at[pl.ds(base, bpw)], idx_v.at[v])
    descs = [
        pltpu.async_copy(tabs[v].at[idx_v.at[v]], rows_v.at[v], sems[v])
        for v in range(4)
    ]
    for v in range(4):
      descs[v].wait()
      pltpu.sync_copy(rows_v.at[v], outs[v].at[pl.ds(base, bpw)])

  return k(item_id, user_segment, region, device_type,
           E_item, E_seg, E_region, E_device)


def _mlp_body(e_i, e_s, e_r, e_d, w1, b1, w2, b2,
              hws, hbs, hwr, hbr, hwd, hbd,
              emb_out, ls_out, lr_out, ld_out):
  x = jnp.dot(e_i[...], w1[0 * _D:1 * _D, :], preferred_element_type=jnp.float32)
  x += jnp.dot(e_s[...], w1[1 * _D:2 * _D, :], preferred_element_type=jnp.float32)
  x += jnp.dot(e_r[...], w1[2 * _D:3 * _D, :], preferred_element_type=jnp.float32)
  x += jnp.dot(e_d[...], w1[3 * _D:4 * _D, :], preferred_element_type=jnp.float32)
  h = jnp.maximum(x + b1[...], 0.0)
  emb = jnp.dot(h, w2[...], preferred_element_type=jnp.float32) + b2[...]
  emb_out[...] = emb
  ls_out[...] = jnp.dot(emb, hws[...], preferred_element_type=jnp.float32) + hbs[...]
  lr_out[...] = jnp.dot(emb, hwr[...], preferred_element_type=jnp.float32) + hbr[...]
  ld_out[...] = jnp.dot(emb, hwd[...], preferred_element_type=jnp.float32) + hbd[...]


def _item_head_body(emb, hwi, hbi, li_out):
  li_out[...] = jnp.dot(emb[...], hwi[...], preferred_element_type=jnp.float32) + hbi[...]


def kernel(item_id, user_segment, region, device_type,
           E_item, E_seg, E_region, E_device,
           W1, b1, W2, b2,
           Hw_item, Hb_item, Hw_seg, Hb_seg,
           Hw_region, Hb_region, Hw_device, Hb_device):
  e_item, e_seg, e_region, e_device = _sc_gather4(
      item_id, user_segment, region, device_type,
      E_item, E_seg, E_region, E_device)

  v_item = Hw_item.shape[1]
  v_seg = Hw_seg.shape[1]
  v_region = Hw_region.shape[1]
  v_device = Hw_device.shape[1]
  n_tiles = pl.cdiv(v_item, _TILE_V)

  emb, ls, lr, ld = pl.pallas_call(
      _mlp_body,
      out_shape=[
          jax.ShapeDtypeStruct((_B, _D), jnp.float32),
          jax.ShapeDtypeStruct((_B, v_seg), jnp.float32),
          jax.ShapeDtypeStruct((_B, v_region), jnp.float32),
          jax.ShapeDtypeStruct((_B, v_device), jnp.float32),
      ],
  )(e_item, e_seg, e_region, e_device,
    W1, b1.reshape(1, _HID), W2, b2.reshape(1, _D),
    Hw_seg, Hb_seg.reshape(1, v_seg),
    Hw_region, Hb_region.reshape(1, v_region),
    Hw_device, Hb_device.reshape(1, v_device))

  li = pl.pallas_call(
      _item_head_body,
      grid=(n_tiles,),
      in_specs=[
          pl.BlockSpec((_B, _D), lambda i: (0, 0)),
          pl.BlockSpec((_D, _TILE_V), lambda i: (0, i)),
          pl.BlockSpec((1, _TILE_V), lambda i: (0, i)),
      ],
      out_specs=pl.BlockSpec((_B, _TILE_V), lambda i: (0, i)),
      out_shape=jax.ShapeDtypeStruct((_B, v_item), jnp.float32),
  )(emb, Hw_item, Hb_item.reshape(1, v_item))

  return (emb, li, ls, lr, ld)


# speedup vs baseline: 2.7091x; 2.7091x over previous
import functools

import jax
import jax.numpy as jnp
from jax import lax
from jax.experimental import pallas as pl
from jax.experimental.pallas import tpu as pltpu
from jax.experimental.pallas import tpu_sc as plsc

_B = 1024
_D = 32
_HID = 128
_TILE_V = 2048


def _sc_gather4(item_id, user_segment, region, device_type,
                E_item, E_seg, E_region, E_device):
  info = plsc.get_sparse_core_info()
  nc, ns = info.num_cores, info.num_subcores
  nw = nc * ns
  bpw = _B // nw

  mesh = plsc.VectorSubcoreMesh(core_axis_name="c", subcore_axis_name="s")
  out_t = [jax.ShapeDtypeStruct((_B, _D), jnp.float32) for _ in range(4)]

  @functools.partial(
      pl.kernel,
      out_type=out_t,
      mesh=mesh,
      compiler_params=pltpu.CompilerParams(use_tc_tiling_on_sc=False),
      scratch_types=[
          pltpu.VMEM((4, bpw), jnp.int32),
          pltpu.VMEM((4, bpw, _D), jnp.float32),
          pltpu.SemaphoreType.DMA,
          pltpu.SemaphoreType.DMA,
          pltpu.SemaphoreType.DMA,
          pltpu.SemaphoreType.DMA,
      ],
  )
  def k(idx0, idx1, idx2, idx3, t0, t1, t2, t3,
        o0, o1, o2, o3, idx_v, rows_v, s0, s1, s2, s3):
    wid = lax.axis_index("s") * nc + lax.axis_index("c")
    base = wid * bpw
    idxs = (idx0, idx1, idx2, idx3)
    tabs = (t0, t1, t2, t3)
    outs = (o0, o1, o2, o3)
    sems = (s0, s1, s2, s3)
    for v in range(4):
      pltpu.sync_copy(idxs[v].at[pl.ds(base, bpw)], idx_v.at[v])
    descs = [
        pltpu.async_copy(tabs[v].at[idx_v.at[v]], rows_v.at[v], sems[v])
        for v in range(4)
    ]
    for v in range(4):
      descs[v].wait()
      pltpu.sync_copy(rows_v.at[v], outs[v].at[pl.ds(base, bpw)])

  return k(item_id, user_segment, region, device_type,
           E_item, E_seg, E_region, E_device)


def _mlp_body(e_i, e_s, e_r, e_d, w1, b1, w2, b2,
              hws, hbs, hwr, hbr, hwd, hbd,
              embT_out, lsT_out, lrT_out, ldT_out):
  x = jnp.dot(e_i[...], w1[0 * _D:1 * _D, :], preferred_element_type=jnp.float32)
  x += jnp.dot(e_s[...], w1[1 * _D:2 * _D, :], preferred_element_type=jnp.float32)
  x += jnp.dot(e_r[...], w1[2 * _D:3 * _D, :], preferred_element_type=jnp.float32)
  x += jnp.dot(e_d[...], w1[3 * _D:4 * _D, :], preferred_element_type=jnp.float32)
  h = jnp.maximum(x + b1[...].reshape(1, _HID), 0.0)
  embT = lax.dot_general(w2[...], h, (((0,), (1,)), ((), ())),
                         preferred_element_type=jnp.float32)
  embT += b2[...][:, None]
  embT_out[...] = embT
  lsT_out[...] = lax.dot_general(hws[...], embT, (((0,), (0,)), ((), ())),
                                 preferred_element_type=jnp.float32) + hbs[...][:, None]
  lrT_out[...] = lax.dot_general(hwr[...], embT, (((0,), (0,)), ((), ())),
                                 preferred_element_type=jnp.float32) + hbr[...][:, None]
  ldT_out[...] = lax.dot_general(hwd[...], embT, (((0,), (0,)), ((), ())),
                                 preferred_element_type=jnp.float32) + hbd[...][:, None]


def _item_head_body(embT, hwi, hbi, liT_out):
  liT_out[...] = lax.dot_general(hwi[...], embT[...], (((0,), (0,)), ((), ())),
                                 preferred_element_type=jnp.float32) + hbi[...][:, None]


def kernel(item_id, user_segment, region, device_type,
           E_item, E_seg, E_region, E_device,
           W1, b1, W2, b2,
           Hw_item, Hb_item, Hw_seg, Hb_seg,
           Hw_region, Hb_region, Hw_device, Hb_device):
  e_item, e_seg, e_region, e_device = _sc_gather4(
      item_id, user_segment, region, device_type,
      E_item, E_seg, E_region, E_device)

  v_item = Hw_item.shape[1]
  v_seg = Hw_seg.shape[1]
  v_region = Hw_region.shape[1]
  v_device = Hw_device.shape[1]
  n_tiles = pl.cdiv(v_item, _TILE_V)

  embT, lsT, lrT, ldT = pl.pallas_call(
      _mlp_body,
      out_shape=[
          jax.ShapeDtypeStruct((_D, _B), jnp.float32),
          jax.ShapeDtypeStruct((v_seg, _B), jnp.float32),
          jax.ShapeDtypeStruct((v_region, _B), jnp.float32),
          jax.ShapeDtypeStruct((v_device, _B), jnp.float32),
      ],
  )(e_item, e_seg, e_region, e_device,
    W1, b1, W2, b2,
    Hw_seg, Hb_seg, Hw_region, Hb_region, Hw_device, Hb_device)

  liT = pl.pallas_call(
      _item_head_body,
      grid=(n_tiles,),
      in_specs=[
          pl.BlockSpec((_D, _B), lambda i: (0, 0)),
          pl.BlockSpec((_D, _TILE_V), lambda i: (0, i)),
          pl.BlockSpec((_TILE_V,), lambda i: (i,)),
      ],
      out_specs=pl.BlockSpec((_TILE_V, _B), lambda i: (i, 0)),
      out_shape=jax.ShapeDtypeStruct((v_item, _B), jnp.float32),
  )(embT, Hw_item, Hb_item)

  return (embT.T, liT.T, lsT.T, lrT.T, ldT.T)


# speedup vs baseline: 3.0205x; 1.1150x over previous
import functools

import jax
import jax.numpy as jnp
from jax import lax
from jax.experimental import pallas as pl
from jax.experimental.pallas import tpu as pltpu
from jax.experimental.pallas import tpu_sc as plsc

_B = 1024
_D = 32
_HID = 128
_TILE_V = 2048
_LANES = 16
_NCHUNK = 8
_CH = _B // _B * 128


def _sc_gather4(item_id, user_segment, region, device_type,
                tflat_item, tflat_seg, tflat_region, tflat_device,
                vocabs):
  info = plsc.get_sparse_core_info()
  nc, ns = info.num_cores, info.num_subcores
  nw = nc * ns
  bpw = _B // nw

  mesh = plsc.VectorSubcoreMesh(core_axis_name="c", subcore_axis_name="s")
  out_t = [jax.ShapeDtypeStruct((nw, _NCHUNK, _CH), jnp.float32)
           for _ in range(4)]

  @functools.partial(
      pl.kernel,
      out_type=out_t,
      mesh=mesh,
      scratch_types=[
          pltpu.VMEM((4, bpw), jnp.int32),
          pltpu.VMEM((4, _NCHUNK, _CH), jnp.int32),
          pltpu.VMEM((4, _NCHUNK, _CH), jnp.float32),
          pltpu.SemaphoreType.DMA,
          pltpu.SemaphoreType.DMA,
          pltpu.SemaphoreType.DMA,
          pltpu.SemaphoreType.DMA,
      ],
  )
  def k(idx0, idx1, idx2, idx3, t0, t1, t2, t3,
        o0, o1, o2, o3, idx_v, bidx, rows, s0, s1, s2, s3):
    wid = lax.axis_index("s") * nc + lax.axis_index("c")
    base = wid * bpw
    idxs = (idx0, idx1, idx2, idx3)
    tabs = (t0, t1, t2, t3)
    outs = (o0, o1, o2, o3)
    sems = (s0, s1, s2, s3)
    for v in range(4):
      pltpu.sync_copy(idxs[v].at[pl.ds(base, bpw)], idx_v.at[v])
    for v in range(4):
      vv = vocabs[v]
      for kk in range(_D * bpw // _LANES):
        r = kk // (bpw // _LANES)
        h = kk % (bpw // _LANES)
        seg = idx_v[v, pl.ds(h * _LANES, _LANES)] + r * vv
        c = (kk * _LANES) // _CH
        off = (kk * _LANES) % _CH
        bidx[v, c, pl.ds(off, _LANES)] = seg
    descs = []
    for v in range(4):
      for c in range(_NCHUNK):
        descs.append(
            pltpu.async_copy(tabs[v].at[bidx.at[v, c]], rows.at[v, c],
                             sems[v]))
    for d in descs:
      d.wait()
    for v in range(4):
      pltpu.sync_copy(rows.at[v], outs[v].at[wid])

  return k(item_id, user_segment, region, device_type,
           tflat_item, tflat_seg, tflat_region, tflat_device)


def _mlp_body(eT_i, eT_s, eT_r, eT_d, w1, b1, w2, b2,
              hws, hbs, hwr, hbr, hwd, hbd,
              embT_out, lsT_out, lrT_out, ldT_out):
  eTs = (eT_i, eT_s, eT_r, eT_d)
  xT = lax.dot_general(w1[0 * _D:1 * _D, :], eT_i[...], (((0,), (0,)), ((), ())),
                       preferred_element_type=jnp.float32)
  for v in range(1, 4):
    xT += lax.dot_general(w1[v * _D:(v + 1) * _D, :], eTs[v][...],
                          (((0,), (0,)), ((), ())),
                          preferred_element_type=jnp.float32)
  hT = jnp.maximum(xT + b1[...][:, None], 0.0)
  embT = lax.dot_general(w2[...], hT, (((0,), (0,)), ((), ())),
                         preferred_element_type=jnp.float32)
  embT += b2[...][:, None]
  embT_out[...] = embT
  lsT_out[...] = lax.dot_general(hws[...], embT, (((0,), (0,)), ((), ())),
                                 preferred_element_type=jnp.float32) + hbs[...][:, None]
  lrT_out[...] = lax.dot_general(hwr[...], embT, (((0,), (0,)), ((), ())),
                                 preferred_element_type=jnp.float32) + hbr[...][:, None]
  ldT_out[...] = lax.dot_general(hwd[...], embT, (((0,), (0,)), ((), ())),
                                 preferred_element_type=jnp.float32) + hbd[...][:, None]


def _item_head_body(embT, hwi, hbi, liT_out):
  liT_out[...] = lax.dot_general(hwi[...], embT[...], (((0,), (0,)), ((), ())),
                                 preferred_element_type=jnp.float32) + hbi[...][:, None]


def kernel(item_id, user_segment, region, device_type,
           E_item, E_seg, E_region, E_device,
           W1, b1, W2, b2,
           Hw_item, Hb_item, Hw_seg, Hb_seg,
           Hw_region, Hb_region, Hw_device, Hb_device):
  vocabs = (E_item.shape[0], E_seg.shape[0], E_region.shape[0],
            E_device.shape[0])
  gathered = _sc_gather4(
      item_id, user_segment, region, device_type,
      E_item.T.reshape(-1), E_seg.T.reshape(-1), E_region.T.reshape(-1),
      E_device.T.reshape(-1), vocabs)
  nw = gathered[0].shape[0]
  bpw = _B // nw
  eTs = [g.reshape(nw, _D, bpw).transpose(1, 0, 2).reshape(_D, _B)
         for g in gathered]

  v_item = Hw_item.shape[1]
  v_seg = Hw_seg.shape[1]
  v_region = Hw_region.shape[1]
  v_device = Hw_device.shape[1]
  n_tiles = pl.cdiv(v_item, _TILE_V)

  embT, lsT, lrT, ldT = pl.pallas_call(
      _mlp_body,
      out_shape=[
          jax.ShapeDtypeStruct((_D, _B), jnp.float32),
          jax.ShapeDtypeStruct((v_seg, _B), jnp.float32),
          jax.ShapeDtypeStruct((v_region, _B), jnp.float32),
          jax.ShapeDtypeStruct((v_device, _B), jnp.float32),
      ],
  )(eTs[0], eTs[1], eTs[2], eTs[3],
    W1, b1, W2, b2,
    Hw_seg, Hb_seg, Hw_region, Hb_region, Hw_device, Hb_device)

  liT = pl.pallas_call(
      _item_head_body,
      grid=(n_tiles,),
      in_specs=[
          pl.BlockSpec((_D, _B), lambda i: (0, 0)),
          pl.BlockSpec((_D, _TILE_V), lambda i: (0, i)),
          pl.BlockSpec((_TILE_V,), lambda i: (i,)),
      ],
      out_specs=pl.BlockSpec((_TILE_V, _B), lambda i: (i, 0)),
      out_shape=jax.ShapeDtypeStruct((v_item, _B), jnp.float32),
  )(embT, Hw_item, Hb_item)

  return (embT.T, liT.T, lsT.T, lrT.T, ldT.T)


# speedup vs baseline: 3.1626x; 1.0470x over previous
import functools

import jax
import jax.numpy as jnp
from jax import lax
from jax.experimental import pallas as pl
from jax.experimental.pallas import tpu as pltpu
from jax.experimental.pallas import tpu_sc as plsc

_B = 1024
_D = 32
_HID = 128
_TILE_V = 2048
_LANES = 16
_NCHUNK = 8
_CH = 128


def _sc_gather(idx_list, tflat_list, vocabs):
  nt = len(tflat_list)
  info = plsc.get_sparse_core_info()
  nc, ns = info.num_cores, info.num_subcores
  nw = nc * ns
  bpw = _B // nw

  mesh = plsc.VectorSubcoreMesh(core_axis_name="c", subcore_axis_name="s")
  out_t = [jax.ShapeDtypeStruct((nw, _NCHUNK, _CH), jnp.float32)
           for _ in range(nt)]

  @functools.partial(
      pl.kernel,
      out_type=out_t,
      mesh=mesh,
      scratch_types=(
          [pltpu.VMEM((nt, bpw), jnp.int32),
           pltpu.VMEM((nt, _NCHUNK, _CH), jnp.int32),
           pltpu.VMEM((nt, _NCHUNK, _CH), jnp.float32)]
          + [pltpu.SemaphoreType.DMA] * (nt + 1)
      ),
  )
  def k(*refs):
    idxs = refs[:nt]
    tabs = refs[nt:2 * nt]
    outs = refs[2 * nt:3 * nt]
    idx_v, bidx, rows = refs[3 * nt:3 * nt + 3]
    isems = refs[3 * nt + 3:3 * nt + 3 + nt]
    gsem = refs[3 * nt + 3 + nt]
    wid = lax.axis_index("s") * nc + lax.axis_index("c")
    base = wid * bpw
    idescs = [
        pltpu.async_copy(idxs[v].at[pl.ds(base, bpw)], idx_v.at[v], isems[v])
        for v in range(nt)
    ]
    gdescs = []
    for v in range(nt):
      idescs[v].wait()
      vv = vocabs[v]
      for kk in range(_D * bpw // _LANES):
        r = kk // (bpw // _LANES)
        h = kk % (bpw // _LANES)
        seg = idx_v[v, pl.ds(h * _LANES, _LANES)] + r * vv
        c = (kk * _LANES) // _CH
        off = (kk * _LANES) % _CH
        bidx[v, c, pl.ds(off, _LANES)] = seg
      for c in range(_NCHUNK):
        gdescs.append(
            pltpu.async_copy(tabs[v].at[bidx.at[v, c]], rows.at[v, c], gsem))
    for d in gdescs:
      d.wait()
    for v in range(nt):
      pltpu.sync_copy(rows.at[v], outs[v].at[wid])

  return k(*idx_list, *tflat_list)


def _fused_body(eT_i, eT_s, eT_r, eT_d, w1, b1, w2, b2,
                hws, hbs, hwr, hbr, hwd, hbd, hwi, hbi,
                liT_out, embT_out, lsT_out, lrT_out, ldT_out):
  step = pl.program_id(0)

  @pl.when(step == 0)
  def _():
    eTs = (eT_i, eT_s, eT_r, eT_d)
    xT = lax.dot_general(w1[0:_D, :], eT_i[...], (((0,), (0,)), ((), ())),
                         preferred_element_type=jnp.float32)
    for v in range(1, 4):
      xT += lax.dot_general(w1[v * _D:(v + 1) * _D, :], eTs[v][...],
                            (((0,), (0,)), ((), ())),
                            preferred_element_type=jnp.float32)
    hT = jnp.maximum(xT + b1[...][:, None], 0.0)
    embT = lax.dot_general(w2[...], hT, (((0,), (0,)), ((), ())),
                           preferred_element_type=jnp.float32)
    embT += b2[...][:, None]
    embT_out[...] = embT
    lsT_out[...] = lax.dot_general(hws[...], embT, (((0,), (0,)), ((), ())),
                                   preferred_element_type=jnp.float32) + hbs[...][:, None]
    lrT_out[...] = lax.dot_general(hwr[...], embT, (((0,), (0,)), ((), ())),
                                   preferred_element_type=jnp.float32) + hbr[...][:, None]
    ldT_out[...] = lax.dot_general(hwd[...], embT, (((0,), (0,)), ((), ())),
                                   preferred_element_type=jnp.float32) + hbd[...][:, None]

  liT_out[...] = lax.dot_general(hwi[...], embT_out[...], (((0,), (0,)), ((), ())),
                                 preferred_element_type=jnp.float32) + hbi[...][:, None]


def kernel(item_id, user_segment, region, device_type,
           E_item, E_seg, E_region, E_device,
           W1, b1, W2, b2,
           Hw_item, Hb_item, Hw_seg, Hb_seg,
           Hw_region, Hb_region, Hw_device, Hb_device):
  g_small = _sc_gather(
      [user_segment, region, device_type],
      [E_seg.T.reshape(-1), E_region.T.reshape(-1), E_device.T.reshape(-1)],
      (E_seg.shape[0], E_region.shape[0], E_device.shape[0]))
  (g_item,) = _sc_gather([item_id], [E_item.T.reshape(-1)],
                         (E_item.shape[0],))
  gathered = [g_item] + list(g_small)

  nw = gathered[0].shape[0]
  bpw = _B // nw
  eTs = [g.reshape(nw, _D, bpw).transpose(1, 0, 2).reshape(_D, _B)
         for g in gathered]

  v_item = Hw_item.shape[1]
  v_seg = Hw_seg.shape[1]
  v_region = Hw_region.shape[1]
  v_device = Hw_device.shape[1]
  n_tiles = pl.cdiv(v_item, _TILE_V)

  const = lambda s: pl.BlockSpec(s, lambda i: tuple(0 for _ in s))

  liT, embT, lsT, lrT, ldT = pl.pallas_call(
      _fused_body,
      grid=(n_tiles,),
      in_specs=[
          const((_D, _B)), const((_D, _B)), const((_D, _B)), const((_D, _B)),
          const((4 * _D, _HID)), const((_HID,)),
          const((_HID, _D)), const((_D,)),
          const((_D, v_seg)), const((v_seg,)),
          const((_D, v_region)), const((v_region,)),
          const((_D, v_device)), const((v_device,)),
          pl.BlockSpec((_D, _TILE_V), lambda i: (0, i)),
          pl.BlockSpec((_TILE_V,), lambda i: (i,)),
      ],
      out_specs=[
          pl.BlockSpec((_TILE_V, _B), lambda i: (i, 0)),
          const((_D, _B)),
          const((v_seg, _B)),
          const((v_region, _B)),
          const((v_device, _B)),
      ],
      out_shape=[
          jax.ShapeDtypeStruct((v_item, _B), jnp.float32),
          jax.ShapeDtypeStruct((_D, _B), jnp.float32),
          jax.ShapeDtypeStruct((v_seg, _B), jnp.float32),
          jax.ShapeDtypeStruct((v_region, _B), jnp.float32),
          jax.ShapeDtypeStruct((v_device, _B), jnp.float32),
      ],
  )(eTs[0], eTs[1], eTs[2], eTs[3],
    W1, b1, W2, b2,
    Hw_seg, Hb_seg, Hw_region, Hb_region, Hw_device, Hb_device,
    Hw_item, Hb_item)

  return (embT.T, liT.T, lsT.T, lrT.T, ldT.T)


# speedup vs baseline: 3.2661x; 1.0327x over previous
import functools

import jax
import jax.numpy as jnp
from jax import lax
from jax.experimental import pallas as pl
from jax.experimental.pallas import tpu as pltpu
from jax.experimental.pallas import tpu_sc as plsc

_B = 1024
_D = 32
_HID = 128
_TILE_V = 2048
_LANES = 16
_NCHUNK = 8
_CH = 128


def _sc_gather(idx_list, tflat, offsets, vocabs):
  nt = len(idx_list)
  info = plsc.get_sparse_core_info()
  nc, ns = info.num_cores, info.num_subcores
  nw = nc * ns
  bpw = _B // nw

  mesh = plsc.VectorSubcoreMesh(core_axis_name="c", subcore_axis_name="s")
  out_t = jax.ShapeDtypeStruct((nw, nt, _NCHUNK, _CH), jnp.float32)

  @functools.partial(
      pl.kernel,
      out_type=out_t,
      mesh=mesh,
      scratch_types=(
          [pltpu.VMEM((nt, bpw), jnp.int32),
           pltpu.VMEM((nt, _NCHUNK, _CH), jnp.int32),
           pltpu.VMEM((nt, _NCHUNK, _CH), jnp.float32)]
          + [pltpu.SemaphoreType.DMA] * (nt + 1)
      ),
  )
  def k(*refs):
    idxs = refs[:nt]
    tab = refs[nt]
    out = refs[nt + 1]
    idx_v, bidx, rows = refs[nt + 2:nt + 5]
    isems = refs[nt + 5:nt + 5 + nt]
    gsem = refs[nt + 5 + nt]
    wid = lax.axis_index("s") * nc + lax.axis_index("c")
    base = wid * bpw
    idescs = [
        pltpu.async_copy(idxs[v].at[pl.ds(base, bpw)], idx_v.at[v], isems[v])
        for v in range(nt)
    ]
    gdescs = []
    for v in range(nt):
      idescs[v].wait()
      vv = vocabs[v]
      for kk in range(_D * bpw // _LANES):
        r = kk // (bpw // _LANES)
        h = kk % (bpw // _LANES)
        seg = idx_v[v, pl.ds(h * _LANES, _LANES)] + (offsets[v] + r * vv)
        c = (kk * _LANES) // _CH
        off = (kk * _LANES) % _CH
        bidx[v, c, pl.ds(off, _LANES)] = seg
      for c in range(_NCHUNK):
        gdescs.append(
            pltpu.async_copy(tab.at[bidx.at[v, c]], rows.at[v, c], gsem))
    for d in gdescs:
      d.wait()
    pltpu.sync_copy(rows, out.at[wid])

  return k(*idx_list, tflat)


def _fused_body(eT_i, eT_small, w1, b1, w2, b2,
                hws, hbs, hwr, hbr, hwd, hbd, hwi, hbi,
                liT_out, embT_out, lsT_out, lrT_out, ldT_out):
  step = pl.program_id(0)

  @pl.when(step == 0)
  def _():
    xT = lax.dot_general(w1[0:_D, :], eT_i[...], (((0,), (0,)), ((), ())),
                         preferred_element_type=jnp.float32)
    for v in range(1, 4):
      xT += lax.dot_general(w1[v * _D:(v + 1) * _D, :], eT_small[v - 1],
                            (((0,), (0,)), ((), ())),
                            preferred_element_type=jnp.float32)
    hT = jnp.maximum(xT + b1[...][:, None], 0.0)
    embT = lax.dot_general(w2[...], hT, (((0,), (0,)), ((), ())),
                           preferred_element_type=jnp.float32)
    embT += b2[...][:, None]
    embT_out[...] = embT
    lsT_out[...] = lax.dot_general(hws[...], embT, (((0,), (0,)), ((), ())),
                                   preferred_element_type=jnp.float32) + hbs[...][:, None]
    lrT_out[...] = lax.dot_general(hwr[...], embT, (((0,), (0,)), ((), ())),
                                   preferred_element_type=jnp.float32) + hbr[...][:, None]
    ldT_out[...] = lax.dot_general(hwd[...], embT, (((0,), (0,)), ((), ())),
                                   preferred_element_type=jnp.float32) + hbd[...][:, None]

  liT_out[...] = lax.dot_general(hwi[...], embT_out[...], (((0,), (0,)), ((), ())),
                                 preferred_element_type=jnp.float32) + hbi[...][:, None]


def kernel(item_id, user_segment, region, device_type,
           E_item, E_seg, E_region, E_device,
           W1, b1, W2, b2,
           Hw_item, Hb_item, Hw_seg, Hb_seg,
           Hw_region, Hb_region, Hw_device, Hb_device):
  vs = (E_seg.shape[0], E_region.shape[0], E_device.shape[0])
  tflat_small = jnp.concatenate(
      [E_seg.T.reshape(-1), E_region.T.reshape(-1), E_device.T.reshape(-1)])
  offs = (0, vs[0] * _D, (vs[0] + vs[1]) * _D)
  g_small = _sc_gather([user_segment, region, device_type],
                       tflat_small, offs, vs)
  g_item = _sc_gather([item_id], E_item.T.reshape(-1), (0,),
                      (E_item.shape[0],))

  nw = g_item.shape[0]
  bpw = _B // nw
  eT_item = (g_item.reshape(nw, _D, bpw).transpose(1, 0, 2)
             .reshape(_D, _B))
  eT_small = (g_small.reshape(nw, 3, _D, bpw).transpose(1, 2, 0, 3)
              .reshape(3, _D, _B))

  v_item = Hw_item.shape[1]
  v_seg = Hw_seg.shape[1]
  v_region = Hw_region.shape[1]
  v_device = Hw_device.shape[1]
  n_tiles = pl.cdiv(v_item, _TILE_V)

  const = lambda s: pl.BlockSpec(s, lambda i: tuple(0 for _ in s))

  liT, embT, lsT, lrT, ldT = pl.pallas_call(
      _fused_body,
      grid=(n_tiles,),
      in_specs=[
          const((_D, _B)), const((3, _D, _B)),
          const((4 * _D, _HID)), const((_HID,)),
          const((_HID, _D)), const((_D,)),
          const((_D, v_seg)), const((v_seg,)),
          const((_D, v_region)), const((v_region,)),
          const((_D, v_device)), const((v_device,)),
          pl.BlockSpec((_D, _TILE_V), lambda i: (0, i)),
          pl.BlockSpec((_TILE_V,), lambda i: (i,)),
      ],
      out_specs=[
          pl.BlockSpec((_TILE_V, _B), lambda i: (i, 0)),
          const((_D, _B)),
          const((v_seg, _B)),
          const((v_region, _B)),
          const((v_device, _B)),
      ],
      out_shape=[
          jax.ShapeDtypeStruct((v_item, _B), jnp.float32),
          jax.ShapeDtypeStruct((_D, _B), jnp.float32),
          jax.ShapeDtypeStruct((v_seg, _B), jnp.float32),
          jax.ShapeDtypeStruct((v_region, _B), jnp.float32),
          jax.ShapeDtypeStruct((v_device, _B), jnp.float32),
      ],
  )(eT_item, eT_small,
    W1, b1, W2, b2,
    Hw_seg, Hb_seg, Hw_region, Hb_region, Hw_device, Hb_device,
    Hw_item, Hb_item)

  return (embT.T, liT.T, lsT.T, lrT.T, ldT.T)
